# Initial kernel scaffold; baseline (speedup 1.0000x reference)
import jax
import jax.numpy as jnp
from jax.experimental import pallas as pl


def kernel(x, table):
    raise NotImplementedError("write your pallas kernel here")


# speedup vs baseline: 4.2841x; 4.2841x over previous
import functools

import jax
import jax.numpy as jnp
from jax.experimental import pallas as pl
from jax.experimental.pallas import tpu as pltpu
from jax.experimental.pallas import tpu_sc as plsc

EMBED = 32
LANES = 128
K = 8


def kernel(x, table):
    B, L = x.shape
    n_idx = B * L
    assert n_idx % (K * LANES) == 0
    x2 = x.reshape(n_idx // LANES, LANES)
    n_steps = n_idx // (K * LANES)
    mesh = plsc.VectorSubcoreMesh(core_axis_name="c", subcore_axis_name="s")

    @functools.partial(
        pl.kernel,
        out_type=jax.ShapeDtypeStruct((n_idx, EMBED), table.dtype),
        mesh=mesh,
        compiler_params=pltpu.CompilerParams(use_tc_tiling_on_sc=False),
    )
    def gather_kernel(table_hbm, idx_hbm, out_hbm):
        def body(idx_vmem, out_vmem):
            @pl.loop(0, K)
            def _(j):
                pltpu.sync_copy(
                    table_hbm.at[idx_vmem.at[j]],
                    out_vmem.at[pl.ds(j * LANES, LANES)],
                )

        pltpu.emit_pipeline(
            body,
            grid=(n_steps,),
            in_specs=[pl.BlockSpec((K, LANES), lambda i: (i, 0))],
            out_specs=[pl.BlockSpec((K * LANES, EMBED), lambda i: (i, 0))],
            core_axis_name=("c", "s"),
            dimension_semantics=(pltpu.PARALLEL,),
        )(idx_hbm, out_hbm)

    return gather_kernel(table, x2).reshape(B, L, EMBED)


# speedup vs baseline: 5.0324x; 1.1747x over previous
import functools

import jax
import jax.numpy as jnp
from jax.experimental import pallas as pl
from jax.experimental.pallas import tpu as pltpu
from jax.experimental.pallas import tpu_sc as plsc

EMBED = 32
LANES = 128
K = 8


def kernel(x, table):
    B, L = x.shape
    n_idx = B * L
    assert n_idx % (K * LANES) == 0
    x2 = x.reshape(n_idx // LANES, LANES)
    n_steps = n_idx // (K * LANES)
    mesh = plsc.VectorSubcoreMesh(core_axis_name="c", subcore_axis_name="s")

    @functools.partial(
        pl.kernel,
        out_type=jax.ShapeDtypeStruct((n_idx, EMBED), table.dtype),
        mesh=mesh,
        scratch_types=[pltpu.SemaphoreType.DMA],
        compiler_params=pltpu.CompilerParams(use_tc_tiling_on_sc=False),
    )
    def gather_kernel(table_hbm, idx_hbm, out_hbm, sem):
        def body(idx_vmem, out_vmem):
            copies = [
                pltpu.async_copy(
                    table_hbm.at[idx_vmem.at[j]],
                    out_vmem.at[pl.ds(j * LANES, LANES)],
                    sem,
                )
                for j in range(K)
            ]
            for c in copies:
                c.wait()

        pltpu.emit_pipeline(
            body,
            grid=(n_steps,),
            in_specs=[pl.BlockSpec((K, LANES), lambda i: (i, 0))],
            out_specs=[pl.BlockSpec((K * LANES, EMBED), lambda i: (i, 0))],
            core_axis_name=("c", "s"),
            dimension_semantics=(pltpu.PARALLEL,),
        )(idx_hbm, out_hbm)

    return gather_kernel(table, x2).reshape(B, L, EMBED)
